# baseline (speedup 1.0000x reference)
import functools

import jax
import jax.numpy as jnp
from jax import lax
from jax.experimental import pallas as pl
from jax.experimental.pallas import tpu as pltpu
from jax.experimental.pallas import tpu_sc as plsc

NC = 2
NS = 16
L = 16
NW = NC * NS

D = 128
C = 128


def _dist_mult_body(si_hbm, pi_hbm, oi_hbm, nodes_hbm, rel_hbm, out_hbm,
                    si_v, pi_v, oi_v, s_rows, p_rows, o_rows, out_v, sem):
    bpw = out_v.shape[0]
    nchunk = bpw // C
    wid = lax.axis_index("s") * NC + lax.axis_index("c")
    base = wid * bpw
    row_ids = lax.iota(jnp.int32, L)

    for c in range(nchunk):
        off = base + c * C
        pltpu.sync_copy(si_hbm.at[pl.ds(off, C)], si_v)
        pltpu.sync_copy(pi_hbm.at[pl.ds(off, C)], pi_v)
        pltpu.sync_copy(oi_hbm.at[pl.ds(off, C)], oi_v)
        cp1 = pltpu.async_copy(nodes_hbm.at[si_v], s_rows, sem)
        cp2 = pltpu.async_copy(rel_hbm.at[pi_v], p_rows, sem)
        cp3 = pltpu.async_copy(nodes_hbm.at[oi_v], o_rows, sem)
        cp1.wait()
        cp2.wait()
        cp3.wait()
        for g in range(C // L):
            rows = row_ids + (g * L)

            def body(d, acc, rows=rows):
                cols = jnp.full((L,), 0, jnp.int32) + d
                sv = plsc.load_gather(s_rows, [rows, cols])
                pv = plsc.load_gather(p_rows, [rows, cols])
                ov = plsc.load_gather(o_rows, [rows, cols])
                return acc + sv * pv * ov

            acc = lax.fori_loop(0, D, body, jnp.zeros((L,), jnp.float32),
                                unroll=4)
            out_v[pl.ds(c * C + g * L, L)] = acc

    pltpu.sync_copy(out_v, out_hbm.at[pl.ds(base, bpw)])


def kernel(triples, nodes, relations):
    b = triples.shape[0]
    bpw = b // NW
    si = triples[:, 0].astype(jnp.int32)
    pi = triples[:, 1].astype(jnp.int32)
    oi = triples[:, 2].astype(jnp.int32)

    mesh = plsc.VectorSubcoreMesh(core_axis_name="c", subcore_axis_name="s")
    run = pl.kernel(
        _dist_mult_body,
        out_type=jax.ShapeDtypeStruct((b,), jnp.float32),
        mesh=mesh,
        compiler_params=pltpu.CompilerParams(needs_layout_passes=False),
        scratch_types=[
            pltpu.VMEM((C,), jnp.int32),
            pltpu.VMEM((C,), jnp.int32),
            pltpu.VMEM((C,), jnp.int32),
            pltpu.VMEM((C, D), jnp.float32),
            pltpu.VMEM((C, D), jnp.float32),
            pltpu.VMEM((C, D), jnp.float32),
            pltpu.VMEM((bpw,), jnp.float32),
            pltpu.SemaphoreType.DMA,
        ],
    )
    return run(si, pi, oi, nodes, relations)
